# baseline (speedup 1.0000x reference)
import functools

import jax
import jax.numpy as jnp
from jax import lax
from jax.experimental import pallas as pl
from jax.experimental.pallas import tpu as pltpu
from jax.experimental.pallas import tpu_sc as plsc

N = 10000
D = 128
T = 4
STEPS = 8
B = 10
NPG = 1000

NC = 2
NS = 16
NW = NC * NS
CH = 128
NCHUNK = 80
NBUF = 2
HC = NCHUNK // 2
E_PAD = NW * CH * NCHUNK
NPAD = 10112
RPT = NPAD // NS

BN = 2000


def _f32_dot(x, w, dims):
    return lax.dot_general(x, w, dims, preferred_element_type=jnp.float32)


def _transform_block(h_ref, w_ref, b_ref, out_ref):
    out_ref[0] = _f32_dot(h_ref[...], w_ref[0],
                           (((1,), (1,)), ((), ()))) + b_ref[0]


def _transform(h, W_et, b_et3):
    return pl.pallas_call(
        _transform_block,
        grid=(T, N // BN),
        in_specs=[
            pl.BlockSpec((BN, D), lambda t, i: (i, 0)),
            pl.BlockSpec((1, D, D), lambda t, i: (t, 0, 0)),
            pl.BlockSpec((1, 1, D), lambda t, i: (t, 0, 0)),
        ],
        out_specs=pl.BlockSpec((1, BN, D), lambda t, i: (t, i, 0)),
        out_shape=jax.ShapeDtypeStruct((T, N, D), jnp.float32),
    )(h, W_et, b_et3)


@functools.partial(
    pl.kernel,
    mesh=plsc.VectorSubcoreMesh(core_axis_name="c", subcore_axis_name="s"),
    out_type=jax.ShapeDtypeStruct((NC, NPAD, D), jnp.float32),
    scratch_types=(
        [
            pltpu.VMEM((HC, CH), jnp.int32),
            pltpu.VMEM((HC, CH), jnp.int32),
            pltpu.VMEM_SHARED((NPAD, D), jnp.float32),
        ]
        + [pltpu.VMEM((CH, D), jnp.float32)] * NBUF
        + [pltpu.SemaphoreType.DMA] * NBUF
    ),
)
def _sc_aggregate(table, idxs, dsts, zrows, out, idx_blk, dst_blk, acc, *bufsem):
    rows = bufsem[:NBUF]
    gsem = bufsem[NBUF:]
    c = lax.axis_index("c")
    s = lax.axis_index("s")
    wid = c * NS + s
    pltpu.sync_copy(zrows.at[pl.ds(s * RPT, RPT)], acc.at[pl.ds(s * RPT, RPT)])
    plsc.subcore_barrier()

    def _gwait(b):
        pltpu.make_async_copy(table.at[idx_blk.at[0]], rows[b], gsem[b]).wait()

    for half in range(2):
        pltpu.sync_copy(idxs.at[wid, pl.ds(half * HC, HC)], idx_blk)
        pltpu.sync_copy(dsts.at[wid, pl.ds(half * HC, HC)], dst_blk)

        for b in range(NBUF):
            pltpu.async_copy(table.at[idx_blk.at[b]], rows[b], gsem[b])

        def chunk(i, carry):
            j0 = i * NBUF
            for b in range(NBUF):
                _gwait(b)
                jn = jnp.minimum(j0 + b + NBUF, HC - 1)
                pltpu.sync_copy(rows[b], acc.at[dst_blk.at[j0 + b]], add=True)
                pltpu.async_copy(table.at[idx_blk.at[jn]], rows[b], gsem[b])
            return carry

        lax.fori_loop(0, HC // NBUF, chunk, 0)
        for b in range(NBUF):
            _gwait(b)
    plsc.subcore_barrier()
    pltpu.sync_copy(acc.at[pl.ds(s * RPT, RPT)], out.at[c, pl.ds(s * RPT, RPT)])


def _gru_block(p0_ref, p1_ref, h_ref, wih_ref, whh_ref, bih_ref, bhh_ref, out_ref):
    a = p0_ref[0] + p1_ref[0]
    h = h_ref[...]
    gi = _f32_dot(a, wih_ref[...], (((1,), (1,)), ((), ()))) + bih_ref[...]
    gh = _f32_dot(h, whh_ref[...], (((1,), (1,)), ((), ()))) + bhh_ref[...]
    r = jax.nn.sigmoid(gi[:, :D] + gh[:, :D])
    z = jax.nn.sigmoid(gi[:, D:2 * D] + gh[:, D:2 * D])
    n = jnp.tanh(gi[:, 2 * D:] + r * gh[:, 2 * D:])
    out_ref[...] = (1.0 - z) * n + z * h


def _gru(parts, h, W_ih, W_hh, bih2, bhh2):
    return pl.pallas_call(
        _gru_block,
        grid=(N // BN,),
        in_specs=[
            pl.BlockSpec((1, BN, D), lambda i: (0, i, 0)),
            pl.BlockSpec((1, BN, D), lambda i: (1, i, 0)),
            pl.BlockSpec((BN, D), lambda i: (i, 0)),
            pl.BlockSpec((3 * D, D), lambda i: (0, 0)),
            pl.BlockSpec((3 * D, D), lambda i: (0, 0)),
            pl.BlockSpec((1, 3 * D), lambda i: (0, 0)),
            pl.BlockSpec((1, 3 * D), lambda i: (0, 0)),
        ],
        out_specs=pl.BlockSpec((BN, D), lambda i: (i, 0)),
        out_shape=jax.ShapeDtypeStruct((N, D), jnp.float32),
    )(parts, parts, h, W_ih, W_hh, bih2, bhh2)


def _cls_block(h_ref, w_ref, b_ref, out_ref):
    pooled = h_ref[...].reshape(B, NPG, D).sum(axis=1)
    ssum = _f32_dot(pooled, w_ref[...], (((1,), (0,)), ((), ())))
    out_ref[...] = jax.nn.sigmoid(ssum + b_ref[0, 0])


def _cls(h, W_cls, b_cls2):
    wrep = jnp.broadcast_to(W_cls.reshape(D, 1), (D, D))
    return pl.pallas_call(
        _cls_block,
        out_shape=jax.ShapeDtypeStruct((B, D), jnp.float32),
    )(h, wrep, b_cls2)


def kernel(features, edge_index, edge_types, W_et, b_et, W_ih, W_hh, b_ih, b_hh,
           W_cls, b_cls):
    src = edge_index[0]
    dst = edge_index[1]
    e = src.shape[0]
    pad = E_PAD - e
    gidx = (edge_types * N + src).astype(jnp.int32)
    pad_g = (jnp.arange(pad, dtype=jnp.int32) * 37) % (T * N)
    pad_d = N + (jnp.arange(pad, dtype=jnp.int32) % (NPAD - N))
    gidx = jnp.concatenate([gidx, pad_g]).reshape(NW, NCHUNK, CH)
    dsts = jnp.concatenate([dst, pad_d]).reshape(NW, NCHUNK, CH)
    zrows = jnp.zeros((NPAD, D), jnp.float32)
    bih2 = b_ih.reshape(1, 3 * D)
    bhh2 = b_hh.reshape(1, 3 * D)
    b_et3 = b_et.reshape(T, 1, D)

    h = features
    for _ in range(STEPS):
        table = _transform(h, W_et, b_et3).reshape(T * N, D)
        parts = _sc_aggregate(table, gidx, dsts, zrows)
        h = _gru(parts, h, W_ih, W_hh, bih2, bhh2)
    out2 = _cls(h, W_cls, b_cls.reshape(1, 1))
    return out2[:, 0]


# speedup vs baseline: 1.1143x; 1.1143x over previous
import functools

import jax
import jax.numpy as jnp
from jax import lax
from jax.experimental import pallas as pl
from jax.experimental.pallas import tpu as pltpu
from jax.experimental.pallas import tpu_sc as plsc

N = 10000
D = 128
T = 4
STEPS = 8
B = 10
NPG = 1000

NC = 2
NS = 16
NW = NC * NS
CH = 128
NCHUNK = 80
NBUF = 2
HC = NCHUNK // 2
E_PAD = NW * CH * NCHUNK
NPAD = 10112
RPT = NPAD // NS

BN = 2000


def _f32_dot(x, w, dims):
    return lax.dot_general(x, w, dims, preferred_element_type=jnp.float32)


def _transform_block(h_ref, w_ref, b_ref, out_ref):
    out_ref[0] = _f32_dot(h_ref[...], w_ref[0],
                           (((1,), (1,)), ((), ()))) + b_ref[0]


def _transform(h, W_et, b_et3):
    return pl.pallas_call(
        _transform_block,
        grid=(T, N // BN),
        in_specs=[
            pl.BlockSpec((BN, D), lambda t, i: (i, 0)),
            pl.BlockSpec((1, D, D), lambda t, i: (t, 0, 0)),
            pl.BlockSpec((1, 1, D), lambda t, i: (t, 0, 0)),
        ],
        out_specs=pl.BlockSpec((1, BN, D), lambda t, i: (t, i, 0)),
        out_shape=jax.ShapeDtypeStruct((T, N, D), jnp.float32),
    )(h, W_et, b_et3)


@functools.partial(
    pl.kernel,
    mesh=plsc.VectorSubcoreMesh(core_axis_name="c", subcore_axis_name="s"),
    out_type=jax.ShapeDtypeStruct((NC, NPAD, D), jnp.float32),
    scratch_types=(
        [
            pltpu.VMEM((HC, CH), jnp.int32),
            pltpu.VMEM((HC, CH), jnp.int32),
            pltpu.VMEM_SHARED((NPAD, D), jnp.float32),
        ]
        + [pltpu.VMEM((CH, D), jnp.float32)] * NBUF
        + [pltpu.SemaphoreType.DMA] * NBUF
    ),
)
def _sc_aggregate(table, idxs, dsts, zrows, out, idx_blk, dst_blk, acc, *bufsem):
    rows = bufsem[:NBUF]
    gsem = bufsem[NBUF:]
    c = lax.axis_index("c")
    s = lax.axis_index("s")
    wid = c * NS + s
    pltpu.sync_copy(zrows.at[pl.ds(s * RPT, RPT)], acc.at[pl.ds(s * RPT, RPT)])
    plsc.subcore_barrier()

    def _gwait(b):
        pltpu.make_async_copy(table.at[idx_blk.at[0]], rows[b], gsem[b]).wait()

    for half in range(2):
        pltpu.sync_copy(idxs.at[wid, pl.ds(half * HC, HC)], idx_blk)
        pltpu.sync_copy(dsts.at[wid, pl.ds(half * HC, HC)], dst_blk)

        for b in range(NBUF):
            pltpu.async_copy(table.at[idx_blk.at[b]], rows[b], gsem[b])

        def chunk(i, carry):
            j0 = i * NBUF
            for b in range(NBUF):
                _gwait(b)
                jn = jnp.minimum(j0 + b + NBUF, HC - 1)
                pltpu.sync_copy(rows[b], acc.at[dst_blk.at[j0 + b]], add=True)
                pltpu.async_copy(table.at[idx_blk.at[jn]], rows[b], gsem[b])
            return carry

        lax.fori_loop(0, HC // NBUF, chunk, 0)
        for b in range(NBUF):
            _gwait(b)
    plsc.subcore_barrier()
    pltpu.sync_copy(acc.at[pl.ds(s * RPT, RPT)], out.at[c, pl.ds(s * RPT, RPT)])


def _gru_block(p0_ref, p1_ref, h_ref, wih_ref, whh_ref, bih_ref, bhh_ref, out_ref):
    a = p0_ref[0] + p1_ref[0]
    h = h_ref[...]
    gi = _f32_dot(a, wih_ref[...], (((1,), (1,)), ((), ()))) + bih_ref[...]
    gh = _f32_dot(h, whh_ref[...], (((1,), (1,)), ((), ()))) + bhh_ref[...]
    r = jax.nn.sigmoid(gi[:, :D] + gh[:, :D])
    z = jax.nn.sigmoid(gi[:, D:2 * D] + gh[:, D:2 * D])
    n = jnp.tanh(gi[:, 2 * D:] + r * gh[:, 2 * D:])
    out_ref[...] = (1.0 - z) * n + z * h


def _gru_tf_block(p0_ref, p1_ref, h_ref, wih_ref, whh_ref, bih_ref, bhh_ref,
                  wet_ref, bet_ref, hout_ref, tout_ref):
    a = p0_ref[0] + p1_ref[0]
    h = h_ref[...]
    gi = _f32_dot(a, wih_ref[...], (((1,), (1,)), ((), ()))) + bih_ref[...]
    gh = _f32_dot(h, whh_ref[...], (((1,), (1,)), ((), ()))) + bhh_ref[...]
    r = jax.nn.sigmoid(gi[:, :D] + gh[:, :D])
    z = jax.nn.sigmoid(gi[:, D:2 * D] + gh[:, D:2 * D])
    n = jnp.tanh(gi[:, 2 * D:] + r * gh[:, 2 * D:])
    hn = (1.0 - z) * n + z * h
    hout_ref[...] = hn
    for t in range(T):
        tout_ref[t] = _f32_dot(hn, wet_ref[t],
                               (((1,), (1,)), ((), ()))) + bet_ref[t]


def _gru_tf(parts, h, W_ih, W_hh, bih2, bhh2, W_et, b_et3):
    return pl.pallas_call(
        _gru_tf_block,
        grid=(N // BN,),
        in_specs=[
            pl.BlockSpec((1, BN, D), lambda i: (0, i, 0)),
            pl.BlockSpec((1, BN, D), lambda i: (1, i, 0)),
            pl.BlockSpec((BN, D), lambda i: (i, 0)),
            pl.BlockSpec((3 * D, D), lambda i: (0, 0)),
            pl.BlockSpec((3 * D, D), lambda i: (0, 0)),
            pl.BlockSpec((1, 3 * D), lambda i: (0, 0)),
            pl.BlockSpec((1, 3 * D), lambda i: (0, 0)),
            pl.BlockSpec((T, D, D), lambda i: (0, 0, 0)),
            pl.BlockSpec((T, 1, D), lambda i: (0, 0, 0)),
        ],
        out_specs=(
            pl.BlockSpec((BN, D), lambda i: (i, 0)),
            pl.BlockSpec((T, BN, D), lambda i: (0, i, 0)),
        ),
        out_shape=(
            jax.ShapeDtypeStruct((N, D), jnp.float32),
            jax.ShapeDtypeStruct((T, N, D), jnp.float32),
        ),
    )(parts, parts, h, W_ih, W_hh, bih2, bhh2, W_et, b_et3)


def _gru(parts, h, W_ih, W_hh, bih2, bhh2):
    return pl.pallas_call(
        _gru_block,
        grid=(N // BN,),
        in_specs=[
            pl.BlockSpec((1, BN, D), lambda i: (0, i, 0)),
            pl.BlockSpec((1, BN, D), lambda i: (1, i, 0)),
            pl.BlockSpec((BN, D), lambda i: (i, 0)),
            pl.BlockSpec((3 * D, D), lambda i: (0, 0)),
            pl.BlockSpec((3 * D, D), lambda i: (0, 0)),
            pl.BlockSpec((1, 3 * D), lambda i: (0, 0)),
            pl.BlockSpec((1, 3 * D), lambda i: (0, 0)),
        ],
        out_specs=pl.BlockSpec((BN, D), lambda i: (i, 0)),
        out_shape=jax.ShapeDtypeStruct((N, D), jnp.float32),
    )(parts, parts, h, W_ih, W_hh, bih2, bhh2)


def _cls_block(h_ref, w_ref, b_ref, out_ref):
    pooled = h_ref[...].reshape(B, NPG, D).sum(axis=1)
    ssum = _f32_dot(pooled, w_ref[...], (((1,), (0,)), ((), ())))
    out_ref[...] = jax.nn.sigmoid(ssum + b_ref[0, 0])


def _cls(h, W_cls, b_cls2):
    wrep = jnp.broadcast_to(W_cls.reshape(D, 1), (D, D))
    return pl.pallas_call(
        _cls_block,
        out_shape=jax.ShapeDtypeStruct((B, D), jnp.float32),
    )(h, wrep, b_cls2)


def kernel(features, edge_index, edge_types, W_et, b_et, W_ih, W_hh, b_ih, b_hh,
           W_cls, b_cls):
    src = edge_index[0]
    dst = edge_index[1]
    e = src.shape[0]
    pad = E_PAD - e
    gidx = (edge_types * N + src).astype(jnp.int32)
    pad_g = (jnp.arange(pad, dtype=jnp.int32) * 37) % (T * N)
    pad_d = N + (jnp.arange(pad, dtype=jnp.int32) % (NPAD - N))
    gidx = jnp.concatenate([gidx, pad_g]).reshape(NW, NCHUNK, CH)
    dsts = jnp.concatenate([dst, pad_d]).reshape(NW, NCHUNK, CH)
    zrows = jnp.zeros((NPAD, D), jnp.float32)
    bih2 = b_ih.reshape(1, 3 * D)
    bhh2 = b_hh.reshape(1, 3 * D)
    b_et3 = b_et.reshape(T, 1, D)

    h = features
    table = _transform(h, W_et, b_et3)
    for step in range(STEPS):
        parts = _sc_aggregate(table.reshape(T * N, D), gidx, dsts, zrows)
        if step < STEPS - 1:
            h, table = _gru_tf(parts, h, W_ih, W_hh, bih2, bhh2, W_et, b_et3)
        else:
            h = _gru(parts, h, W_ih, W_hh, bih2, bhh2)
    out2 = _cls(h, W_cls, b_cls.reshape(1, 1))
    return out2[:, 0]
